# baseline (speedup 1.0000x reference)
import functools

import jax
import jax.numpy as jnp
from jax import lax
from jax.experimental import pallas as pl
from jax.experimental.pallas import tpu as pltpu
from jax.experimental.pallas import tpu_sc as plsc

_B = 16384
_F = 26
_ROWS = 100001
_D = 32

_NC = 2
_NS = 16
_NW = _NC * _NS
_BPW = _B // _NW
_CB = 8
_NCHUNKS = _BPW // _CB

_mesh = plsc.VectorSubcoreMesh(core_axis_name="c", subcore_axis_name="s")


@functools.partial(
    pl.kernel,
    mesh=_mesh,
    out_type=jax.ShapeDtypeStruct((_B, _F, _D), jnp.float32),
    scratch_types=[
        pltpu.VMEM((2, _CB, 32), jnp.int32),
        pltpu.VMEM((2, _CB, _F, _D), jnp.float32),
        pltpu.SemaphoreType.DMA,
        pltpu.SemaphoreType.DMA,
        pltpu.SemaphoreType.DMA,
        pltpu.SemaphoreType.DMA,
        pltpu.SemaphoreType.DMA,
    ],
)
def _embed(x_hbm, tab_hbm, out_hbm, xv, rows_v, xsem, gs0, gs1, os0, os1):
    wid = lax.axis_index("s") * _NC + lax.axis_index("c")
    base = wid * _BPW
    gsems = (gs0, gs1)
    osems = (os0, os1)

    def stage_x(c, buf):
        b0 = base + c * _CB
        for j in range(_CB):
            pltpu.async_copy(
                x_hbm.at[b0 + j, :], xv.at[buf, j, pl.ds(0, _F)], xsem
            )

    def wait_x():
        def body(k, carry):
            pltpu.make_async_copy(
                x_hbm.at[0, :], xv.at[0, 0, pl.ds(0, _F)], xsem
            ).wait()
            return carry

        lax.fori_loop(0, _CB, body, 0)

    def issue_gathers(c, buf):
        b0 = base + c * _CB
        del b0
        for j in range(_CB):
            va = xv[buf, j, pl.ds(0, 16)]
            vb = xv[buf, j, pl.ds(16, 16)]
            for f in range(_F):
                idx = va[f] if f < 16 else vb[f - 16]
                pltpu.async_copy(
                    tab_hbm.at[f, pl.ds(idx, 1), :],
                    rows_v.at[buf, j, pl.ds(f, 1), :],
                    gsems[buf],
                )

    def wait_gathers(buf):
        def body(k, carry):
            pltpu.make_async_copy(
                tab_hbm.at[0, pl.ds(0, 1), :],
                rows_v.at[0, 0, pl.ds(0, 1), :],
                gsems[buf],
            ).wait()
            return carry

        lax.fori_loop(0, _CB * _F, body, 0)

    def issue_writes(c, buf):
        b0 = base + c * _CB

        def body(k, carry):
            j = k // _F
            f = lax.rem(k, _F)
            pltpu.async_copy(
                rows_v.at[buf, j, pl.ds(f, 1), :],
                out_hbm.at[b0 + j, pl.ds(f, 1), :],
                osems[buf],
            )
            return carry

        lax.fori_loop(0, _CB * _F, body, 0)

    def wait_writes(buf):
        def body(k, carry):
            pltpu.make_async_copy(
                rows_v.at[0, 0, pl.ds(0, 1), :],
                out_hbm.at[0, pl.ds(0, 1), :],
                osems[buf],
            ).wait()
            return carry

        lax.fori_loop(0, _CB * _F, body, 0)

    def chunk_step(c, buf):
        nbuf = 1 - buf

        @pl.when(c + 1 < _NCHUNKS)
        def _():
            stage_x(c + 1, nbuf)

        @pl.when(c >= 1)
        def _():
            wait_writes(nbuf)

        @pl.when(c + 1 < _NCHUNKS)
        def _():
            wait_x()
            issue_gathers(c + 1, nbuf)

        wait_gathers(buf)
        issue_writes(c, buf)

    stage_x(0, 0)
    wait_x()
    issue_gathers(0, 0)

    def pair_body(c2, carry):
        chunk_step(2 * c2, 0)
        chunk_step(2 * c2 + 1, 1)
        return carry

    lax.fori_loop(0, _NCHUNKS // 2, pair_body, 0)
    wait_writes(1)


def kernel(x, tables):
    return _embed(x, tables)


# speedup vs baseline: 1.0543x; 1.0543x over previous
import functools

import jax
import jax.numpy as jnp
from jax import lax
from jax.experimental import pallas as pl
from jax.experimental.pallas import tpu as pltpu
from jax.experimental.pallas import tpu_sc as plsc

_B = 16384
_F = 26
_ROWS = 100001
_D = 32

_NC = 2
_NS = 16
_NW = _NC * _NS
_BPW = _B // _NW
_CB = 8
_NCHUNKS = _BPW // _CB

_mesh = plsc.VectorSubcoreMesh(core_axis_name="c", subcore_axis_name="s")


@functools.partial(
    pl.kernel,
    mesh=_mesh,
    out_type=jax.ShapeDtypeStruct((_B, _F, _D), jnp.float32),
    scratch_types=[
        pltpu.VMEM((2, _CB, 32), jnp.int32),
        pltpu.VMEM((2, _CB, _F, _D), jnp.float32),
        pltpu.SemaphoreType.DMA,
        pltpu.SemaphoreType.DMA,
        pltpu.SemaphoreType.DMA,
        pltpu.SemaphoreType.DMA,
        pltpu.SemaphoreType.DMA,
    ],
)
def _embed(x_hbm, tab_hbm, out_hbm, xv, rows_v, xsem, gs0, gs1, os0, os1):
    wid = lax.axis_index("s") * _NC + lax.axis_index("c")
    base = wid * _BPW
    gsems = (gs0, gs1)
    osems = (os0, os1)

    def stage_x(c, buf):
        b0 = base + c * _CB
        for j in range(_CB):
            pltpu.async_copy(
                x_hbm.at[b0 + j, :], xv.at[buf, j, pl.ds(0, _F)], xsem
            )

    def wait_x():
        def body(k, carry):
            pltpu.make_async_copy(
                x_hbm.at[0, :], xv.at[0, 0, pl.ds(0, _F)], xsem
            ).wait()
            return carry

        lax.fori_loop(0, _CB, body, 0)

    def issue_gathers(c, buf):
        b0 = base + c * _CB
        del b0
        for j in range(_CB):
            va = xv[buf, j, pl.ds(0, 16)]
            vb = xv[buf, j, pl.ds(16, 16)]
            for f in range(_F):
                idx = va[f] if f < 16 else vb[f - 16]
                pltpu.async_copy(
                    tab_hbm.at[f, pl.ds(idx, 1), :],
                    rows_v.at[buf, j, pl.ds(f, 1), :],
                    gsems[buf],
                )

    def wait_gathers(buf):
        def body(k, carry):
            pltpu.make_async_copy(
                tab_hbm.at[0, pl.ds(0, 1), :],
                rows_v.at[0, 0, pl.ds(0, 1), :],
                gsems[buf],
            ).wait()
            return carry

        lax.fori_loop(0, _CB * _F, body, 0)

    def issue_writes(c, buf):
        b0 = base + c * _CB
        for j in range(_CB):
            pltpu.async_copy(
                rows_v.at[buf, j], out_hbm.at[b0 + j], osems[buf]
            )

    def wait_writes(buf):
        def body(k, carry):
            pltpu.make_async_copy(
                rows_v.at[0, 0], out_hbm.at[0], osems[buf]
            ).wait()
            return carry

        lax.fori_loop(0, _CB, body, 0)

    def chunk_step(c, buf):
        nbuf = 1 - buf

        @pl.when(c + 1 < _NCHUNKS)
        def _():
            stage_x(c + 1, nbuf)

        @pl.when(c >= 1)
        def _():
            wait_writes(nbuf)

        @pl.when(c + 1 < _NCHUNKS)
        def _():
            wait_x()
            issue_gathers(c + 1, nbuf)

        wait_gathers(buf)
        issue_writes(c, buf)

    stage_x(0, 0)
    wait_x()
    issue_gathers(0, 0)

    def pair_body(c2, carry):
        chunk_step(2 * c2, 0)
        chunk_step(2 * c2 + 1, 1)
        return carry

    lax.fori_loop(0, _NCHUNKS // 2, pair_body, 0)
    wait_writes(1)


def kernel(x, tables):
    return _embed(x, tables)
